# Initial kernel scaffold; baseline (speedup 1.0000x reference)
import jax
import jax.numpy as jnp
from jax.experimental import pallas as pl


def kernel(x, W_conv, b_conv, W_td, b_td, W_bu, b_bu, W_g2, b_g2, W_fc, b_fc):
    raise NotImplementedError("write your pallas kernel here")


# speedup vs baseline: 1.3447x; 1.3447x over previous
import jax
import jax.numpy as jnp
import numpy as np
from jax.experimental import pallas as pl

B = 64
IN_FEATS = 64
NUM_NODES = 16
D_NODE = 4
HID = 128
H = W = 224
WPAD = 256
HPAD = 232


def _fused_kernel(x_ref, w_ref, b_ref, d_ref, sel_ref, atd_ref, abu_ref,
                  wtd_ref, btd_ref, wbu_ref, bbu_ref, wg2_ref, bg2_ref,
                  wfc_ref, bfc_ref, out_ref):
    def mm(a, b):
        return jax.lax.dot_general(a, b, (((1,), (0,)), ((), ())),
                                   preferred_element_type=jnp.float32)

    def body(c, acc):
        xs_blk = x_ref[0, :, pl.ds(c * 8, 16), :]
        for dh in range(8):
            pieces = []
            for kh in range(3):
                xr = xs_blk[:, dh + kh, :]
                for kw in range(3):
                    pieces.append(xr[:, kw:kw + WPAD])
            p = jnp.concatenate(pieces, axis=0)
            r = mm(w_ref[:], p)
            acc = acc + jnp.maximum(r + b_ref[:], 0.0)
        return acc

    acc = jax.lax.fori_loop(0, H // 8, body,
                            jnp.zeros((IN_FEATS, WPAD), jnp.float32))
    mask = (jax.lax.broadcasted_iota(jnp.int32, (1, WPAD), 1) < W)
    acc = jnp.where(mask, acc, 0.0)
    pooled = jnp.sum(acc, axis=1, keepdims=True) * (1.0 / (H * W))

    hs = mm(sel_ref[:], pooled * d_ref[:])
    tdn = mm(atd_ref[:], hs)
    bun = mm(abu_ref[:], hs)
    td = jnp.maximum(mm(tdn, wtd_ref[:]) + btd_ref[:], 0.0)
    bu = jnp.maximum(mm(bun, wbu_ref[:]) + bbu_ref[:], 0.0)
    z = jnp.concatenate([td, bu], axis=1)
    z2 = jnp.maximum(mm(mm(atd_ref[:], z), wg2_ref[:]) + bg2_ref[:], 0.0)
    g = jnp.sum(z2, axis=0, keepdims=True) * (1.0 / NUM_NODES)
    out_ref[0] = mm(g, wfc_ref[:]) + bfc_ref[:]


def kernel(x, W_conv, b_conv, W_td, b_td, W_bu, b_bu, W_g2, b_g2, W_fc, b_fc):
    xp = jnp.pad(x, ((0, 0), (0, 0), (1, HPAD - H - 1), (1, WPAD + 2 - W - 1)))
    w2 = W_conv.transpose(0, 2, 3, 1).reshape(IN_FEATS, 27)
    bc = b_conv.reshape(IN_FEATS, 1)

    f = np.arange(IN_FEATS)
    dmat = jnp.asarray((f[:, None] % D_NODE) == np.arange(D_NODE)[None, :],
                       jnp.float32)
    sel = jnp.asarray((f[None, :] // D_NODE) == np.arange(NUM_NODES)[:, None],
                      jnp.float32)

    idx = np.arange(NUM_NODES)
    atd = jnp.asarray(np.where(idx[None, :] > idx[:, None],
                               1.0 / np.maximum(NUM_NODES - 1 - idx, 1)[:, None],
                               0.0), jnp.float32)
    abu = jnp.asarray(np.where(idx[None, :] < idx[:, None],
                               1.0 / np.maximum(idx, 1)[:, None],
                               0.0), jnp.float32)

    num_classes = W_fc.shape[1]
    full = lambda shape: pl.BlockSpec(shape, lambda i: tuple(0 for _ in shape))
    out = pl.pallas_call(
        _fused_kernel,
        grid=(B,),
        in_specs=[
            pl.BlockSpec((1, 3, HPAD, WPAD + 2), lambda i: (i, 0, 0, 0)),
            full((IN_FEATS, 27)),
            full((IN_FEATS, 1)),
            full((IN_FEATS, D_NODE)),
            full((NUM_NODES, IN_FEATS)),
            full((NUM_NODES, NUM_NODES)),
            full((NUM_NODES, NUM_NODES)),
            full((D_NODE, HID)),
            full((1, HID)),
            full((D_NODE, HID)),
            full((1, HID)),
            full((2 * HID, HID)),
            full((1, HID)),
            full((HID, num_classes)),
            full((1, num_classes)),
        ],
        out_specs=pl.BlockSpec((1, 1, num_classes), lambda i: (i, 0, 0)),
        out_shape=jax.ShapeDtypeStruct((B, 1, num_classes), jnp.float32),
    )(xp, w2, bc, dmat, sel, atd, abu, W_td, b_td.reshape(1, HID), W_bu,
      b_bu.reshape(1, HID), W_g2, b_g2.reshape(1, HID), W_fc,
      b_fc.reshape(1, num_classes))
    return out.reshape(B, num_classes)


# speedup vs baseline: 1.4291x; 1.0628x over previous
import jax
import jax.numpy as jnp
import numpy as np
from jax.experimental import pallas as pl
from jax.experimental.pallas import tpu as pltpu

B = 64
IN_FEATS = 64
NUM_NODES = 16
D_NODE = 4
HID = 128
H = W = 224
WPAD = 256
HPAD = 232


def _fused_kernel(x_ref, w_ref, b_ref, d_ref, sel_ref, atd_ref, abu_ref,
                  wtd_ref, btd_ref, wbu_ref, bbu_ref, wg2_ref, bg2_ref,
                  wfc_ref, bfc_ref, out_ref):
    def mm(a, b):
        return jax.lax.dot_general(a, b, (((1,), (0,)), ((), ())),
                                   preferred_element_type=jnp.float32)

    def body(c, acc):
        xs_blk = x_ref[0, :, pl.ds(c * 8, 16), :]
        p = jnp.concatenate(
            [xs_blk[:, :, kw:kw + WPAD].reshape(48, WPAD) for kw in range(3)],
            axis=0)
        r = mm(w_ref[:], p)
        r = jnp.maximum(r + b_ref[:], 0.0)
        return acc + jnp.sum(r.reshape(8, IN_FEATS, WPAD), axis=0)

    acc = jax.lax.fori_loop(0, H // 8, body,
                            jnp.zeros((IN_FEATS, WPAD), jnp.float32))
    mask = (jax.lax.broadcasted_iota(jnp.int32, (1, WPAD), 1) < W)
    acc = jnp.where(mask, acc, 0.0)
    pooled = jnp.sum(acc, axis=1, keepdims=True) * (1.0 / (H * W))

    hs = mm(sel_ref[:], pooled * d_ref[:])
    tdn = mm(atd_ref[:], hs)
    bun = mm(abu_ref[:], hs)
    td = jnp.maximum(mm(tdn, wtd_ref[:]) + btd_ref[:], 0.0)
    bu = jnp.maximum(mm(bun, wbu_ref[:]) + bbu_ref[:], 0.0)
    z = jnp.concatenate([td, bu], axis=1)
    z2 = jnp.maximum(mm(mm(atd_ref[:], z), wg2_ref[:]) + bg2_ref[:], 0.0)
    g = jnp.sum(z2, axis=0, keepdims=True) * (1.0 / NUM_NODES)
    out_ref[0] = mm(g, wfc_ref[:]) + bfc_ref[:]


def kernel(x, W_conv, b_conv, W_td, b_td, W_bu, b_bu, W_g2, b_g2, W_fc, b_fc):
    xp = jnp.pad(x, ((0, 0), (0, 0), (1, HPAD - H - 1), (1, WPAD + 2 - W - 1)))
    shift = ((np.arange(16)[None, :, None] - np.arange(8)[:, None, None])
             == np.arange(3)[None, None, :]).astype(np.float32)
    w2 = jnp.einsum('oihw,drh->dowir', W_conv,
                    jnp.asarray(shift)).reshape(8 * IN_FEATS, 144)
    bc = jnp.tile(b_conv, 8).reshape(8 * IN_FEATS, 1)

    f = np.arange(IN_FEATS)
    dmat = jnp.asarray((f[:, None] % D_NODE) == np.arange(D_NODE)[None, :],
                       jnp.float32)
    sel = jnp.asarray((f[None, :] // D_NODE) == np.arange(NUM_NODES)[:, None],
                      jnp.float32)

    idx = np.arange(NUM_NODES)
    atd = jnp.asarray(np.where(idx[None, :] > idx[:, None],
                               1.0 / np.maximum(NUM_NODES - 1 - idx, 1)[:, None],
                               0.0), jnp.float32)
    abu = jnp.asarray(np.where(idx[None, :] < idx[:, None],
                               1.0 / np.maximum(idx, 1)[:, None],
                               0.0), jnp.float32)

    num_classes = W_fc.shape[1]
    full = lambda shape: pl.BlockSpec(shape, lambda i: tuple(0 for _ in shape))
    out = pl.pallas_call(
        _fused_kernel,
        grid=(B,),
        in_specs=[
            pl.BlockSpec((1, 3, HPAD, WPAD + 2), lambda i: (i, 0, 0, 0)),
            full((8 * IN_FEATS, 144)),
            full((8 * IN_FEATS, 1)),
            full((IN_FEATS, D_NODE)),
            full((NUM_NODES, IN_FEATS)),
            full((NUM_NODES, NUM_NODES)),
            full((NUM_NODES, NUM_NODES)),
            full((D_NODE, HID)),
            full((1, HID)),
            full((D_NODE, HID)),
            full((1, HID)),
            full((2 * HID, HID)),
            full((1, HID)),
            full((HID, num_classes)),
            full((1, num_classes)),
        ],
        out_specs=pl.BlockSpec((1, 1, num_classes), lambda i: (i, 0, 0)),
        out_shape=jax.ShapeDtypeStruct((B, 1, num_classes), jnp.float32),
        compiler_params=pltpu.CompilerParams(
            dimension_semantics=("parallel",)),
    )(xp, w2, bc, dmat, sel, atd, abu, W_td, b_td.reshape(1, HID), W_bu,
      b_bu.reshape(1, HID), W_g2, b_g2.reshape(1, HID), W_fc,
      b_fc.reshape(1, num_classes))
    return out.reshape(B, num_classes)
